# baseline (speedup 1.0000x reference)
import functools

import jax
import jax.numpy as jnp
from jax import lax
from jax.experimental import pallas as pl
from jax.experimental.pallas import tpu as pltpu
from jax.experimental.pallas import tpu_sc as plsc

_K = 5
_NEG = float("-inf")
_NS = 16
_LANE = 16


def _insert(regs, cur):
    for u in range(_K):
        hi = jnp.maximum(regs[u], cur)
        cur = jnp.minimum(regs[u], cur)
        regs[u] = hi


def _seed(chunks):
    regs = list(chunks)
    for t in range(1, _K):
        cur = regs[t]
        for u in range(t):
            hi = jnp.maximum(regs[u], cur)
            cur = jnp.minimum(regs[u], cur)
            regs[u] = hi
        regs[t] = cur
    return regs


def _loss_body(x_ref, lrow_ref, lab_ref, out_ref, acc_ref):
    i = pl.program_id(0)

    @pl.when(i == 0)
    def _init():
        acc_ref[0] = jnp.float32(0.0)
        acc_ref[1] = jnp.float32(0.0)

    x = x_ref[...]
    lrow = lrow_ref[...]
    lab = lab_ref[...]
    r, v_dim = x.shape
    nchunk = v_dim // 128

    nw = 4
    chunk = [x[:, c * 128:(c + 1) * 128] for c in range(nw * _K)]
    nets = [_seed(chunk[w::nw]) for w in range(nw)]
    ezs = [jnp.exp(chunk[w]) for w in range(nw)]
    for c in chunk[nw:]:
        ezs = ezs[1:] + [ezs[0] + jnp.exp(c)]
    for c in range(nw * _K, nchunk, nw):
        for w in range(nw):
            cw = x[:, (c + w) * 128:(c + w + 1) * 128]
            _insert(nets[w], cw)
            ezs[w] = ezs[w] + jnp.exp(cw)

    cand = jnp.concatenate(sum(nets, []), axis=1)
    top = jnp.max(cand, axis=1, keepdims=True)
    vs = [top]
    for _ in range(_K - 1):
        top = jnp.max(jnp.where(cand >= top, _NEG, cand),
                      axis=1, keepdims=True)
        vs.append(top)

    ez = ezs[0]
    for e in ezs[1:]:
        ez = ez + e
    z = jnp.sum(ez, axis=1, keepdims=True)

    col = lax.broadcasted_iota(jnp.int32, (r, 128), 1)
    ll = jnp.max(jnp.where(col == (lab & 127), lrow, _NEG),
                 axis=1, keepdims=True)

    pp = jnp.exp(ll) / z
    e0 = jnp.exp(pp)
    s = e0
    kc = jnp.zeros((r, 1), jnp.float32)
    for vt in vs:
        keep = vt != ll
        tv = jnp.exp(vt) / z
        s = s + jnp.where(keep, jnp.exp(tv), 0.0)
        kc = kc + jnp.where(keep, 1.0, 0.0)
    mini0 = e0 / s
    per = (1.0 - (kc + 1.0) * mini0) / kc
    validf = ((lab != 0) & (vs[0] != ll)).astype(jnp.float32)
    acc_ref[0] += jnp.sum(per * validf)
    acc_ref[1] += jnp.sum(validf)

    @pl.when(i == pl.num_programs(0) - 1)
    def _fin():
        cnt = acc_ref[1]
        out_ref[0, 0] = jnp.where(cnt > 0.0,
                                  acc_ref[0] / jnp.maximum(cnt, 1.0),
                                  jnp.float32(0.0))


def _gather_body(lab_hbm, table_hbm, out_hbm, lab_v, idx_v, rows_v, sem):
    sid = lax.axis_index("s")
    base = sid * 64
    lanes = lax.iota(jnp.int32, _LANE)
    pltpu.sync_copy(lab_hbm.at[pl.ds(base, 64)], lab_v)
    for c in range(4):
        lab16 = lab_v[pl.ds(c * 16, 16)]
        pos = base + c * 16 + lanes
        idx_v[pl.ds(c * 16, 16)] = pos * 256 + (lab16 >> 7)
    pltpu.async_copy(table_hbm.at[idx_v], rows_v, sem).wait()
    pltpu.sync_copy(rows_v, out_hbm.at[pl.ds(base, 64)])


def kernel(label_ids, logits):
    b, s, v = logits.shape
    n = b * s
    x = logits.reshape(n, v)
    labf = label_ids.reshape(n)
    lab2 = label_ids.reshape(n, 1)
    table = logits.reshape(n * (v // 128), 128)
    r = 16

    mesh = plsc.VectorSubcoreMesh(core_axis_name="c", subcore_axis_name="s",
                                  num_cores=1)

    gather = pl.kernel(
        _gather_body,
        mesh=mesh,
        out_type=jax.ShapeDtypeStruct((n, 128), jnp.float32),
        scratch_types=[
            pltpu.VMEM((64,), jnp.int32),
            pltpu.VMEM((64,), jnp.int32),
            pltpu.VMEM((64, 128), jnp.float32),
            pltpu.SemaphoreType.DMA,
        ],
    )
    lrows = gather(labf, table)

    out = pl.pallas_call(
        _loss_body,
        grid=(n // r,),
        in_specs=[
            pl.BlockSpec((r, v), lambda i: (i, 0)),
            pl.BlockSpec((r, 128), lambda i: (i, 0)),
            pl.BlockSpec((r, 1), lambda i: (i, 0)),
        ],
        out_specs=pl.BlockSpec(memory_space=pltpu.SMEM),
        out_shape=jax.ShapeDtypeStruct((1, 1), jnp.float32),
        scratch_shapes=[pltpu.SMEM((2,), jnp.float32)],
    )(x, lrows, lab2)
    return out[0, 0]


# speedup vs baseline: 1.0101x; 1.0101x over previous
import functools

import jax
import jax.numpy as jnp
from jax import lax
from jax.experimental import pallas as pl
from jax.experimental.pallas import tpu as pltpu
from jax.experimental.pallas import tpu_sc as plsc

_K = 5
_NEG = float("-inf")
_NS = 16
_LANE = 16


def _insert(regs, cur):
    for u in range(_K):
        hi = jnp.maximum(regs[u], cur)
        cur = jnp.minimum(regs[u], cur)
        regs[u] = hi


def _seed(chunks):
    regs = list(chunks)
    for t in range(1, _K):
        cur = regs[t]
        for u in range(t):
            hi = jnp.maximum(regs[u], cur)
            cur = jnp.minimum(regs[u], cur)
            regs[u] = hi
        regs[t] = cur
    return regs


def _loss_body(x_ref, lrow_ref, lab_ref, out_ref, acc_ref):
    i = pl.program_id(0)

    @pl.when(i == 0)
    def _init():
        acc_ref[0] = jnp.float32(0.0)
        acc_ref[1] = jnp.float32(0.0)

    x = x_ref[...]
    lrow = lrow_ref[...]
    lab = lab_ref[...]
    r, v_dim = x.shape
    nchunk = v_dim // 128

    nw = 2
    chunk = [x[:, c * 128:(c + 1) * 128] for c in range(nw * _K)]
    nets = [_seed(chunk[w::nw]) for w in range(nw)]
    ezs = [jnp.exp(chunk[w]) for w in range(nw)]
    for c in chunk[nw:]:
        ezs = ezs[1:] + [ezs[0] + jnp.exp(c)]
    for c in range(nw * _K, nchunk, nw):
        for w in range(nw):
            cw = x[:, (c + w) * 128:(c + w + 1) * 128]
            _insert(nets[w], cw)
            ezs[w] = ezs[w] + jnp.exp(cw)

    cand = jnp.concatenate(sum(nets, []), axis=1)
    top = jnp.max(cand, axis=1, keepdims=True)
    vs = [top]
    for _ in range(_K - 1):
        top = jnp.max(jnp.where(cand >= top, _NEG, cand),
                      axis=1, keepdims=True)
        vs.append(top)

    ez = ezs[0]
    for e in ezs[1:]:
        ez = ez + e
    z = jnp.sum(ez, axis=1, keepdims=True)

    col = lax.broadcasted_iota(jnp.int32, (r, 128), 1)
    ll = jnp.max(jnp.where(col == (lab & 127), lrow, _NEG),
                 axis=1, keepdims=True)

    pp = jnp.exp(ll) / z
    e0 = jnp.exp(pp)
    s = e0
    kc = jnp.zeros((r, 1), jnp.float32)
    for vt in vs:
        keep = vt != ll
        tv = jnp.exp(vt) / z
        s = s + jnp.where(keep, jnp.exp(tv), 0.0)
        kc = kc + jnp.where(keep, 1.0, 0.0)
    mini0 = e0 / s
    per = (1.0 - (kc + 1.0) * mini0) / kc
    validf = ((lab != 0) & (vs[0] != ll)).astype(jnp.float32)
    acc_ref[0] += jnp.sum(per * validf)
    acc_ref[1] += jnp.sum(validf)

    @pl.when(i == pl.num_programs(0) - 1)
    def _fin():
        cnt = acc_ref[1]
        out_ref[0, 0] = jnp.where(cnt > 0.0,
                                  acc_ref[0] / jnp.maximum(cnt, 1.0),
                                  jnp.float32(0.0))


def _gather_body(lab_hbm, table_hbm, out_hbm, lab_v, idx_v, rows_v, sem):
    sid = lax.axis_index("s")
    base = sid * 64
    lanes = lax.iota(jnp.int32, _LANE)
    pltpu.sync_copy(lab_hbm.at[pl.ds(base, 64)], lab_v)
    for c in range(4):
        lab16 = lab_v[pl.ds(c * 16, 16)]
        pos = base + c * 16 + lanes
        idx_v[pl.ds(c * 16, 16)] = pos * 256 + (lab16 >> 7)
    pltpu.async_copy(table_hbm.at[idx_v], rows_v, sem).wait()
    pltpu.sync_copy(rows_v, out_hbm.at[pl.ds(base, 64)])


def kernel(label_ids, logits):
    b, s, v = logits.shape
    n = b * s
    x = logits.reshape(n, v)
    labf = label_ids.reshape(n)
    lab2 = label_ids.reshape(n, 1)
    table = logits.reshape(n * (v // 128), 128)
    r = 16

    mesh = plsc.VectorSubcoreMesh(core_axis_name="c", subcore_axis_name="s",
                                  num_cores=1)

    gather = pl.kernel(
        _gather_body,
        mesh=mesh,
        out_type=jax.ShapeDtypeStruct((n, 128), jnp.float32),
        scratch_types=[
            pltpu.VMEM((64,), jnp.int32),
            pltpu.VMEM((64,), jnp.int32),
            pltpu.VMEM((64, 128), jnp.float32),
            pltpu.SemaphoreType.DMA,
        ],
    )
    lrows = gather(labf, table)

    out = pl.pallas_call(
        _loss_body,
        grid=(n // r,),
        in_specs=[
            pl.BlockSpec((r, v), lambda i: (i, 0)),
            pl.BlockSpec((r, 128), lambda i: (i, 0)),
            pl.BlockSpec((r, 1), lambda i: (i, 0)),
        ],
        out_specs=pl.BlockSpec(memory_space=pltpu.SMEM),
        out_shape=jax.ShapeDtypeStruct((1, 1), jnp.float32),
        scratch_shapes=[pltpu.SMEM((2,), jnp.float32)],
    )(x, lrows, lab2)
    return out[0, 0]
